# baseline (speedup 1.0000x reference)
import functools
import math

import jax
import jax.numpy as jnp
from jax import lax
from jax.experimental import pallas as pl
from jax.experimental.pallas import tpu as pltpu
from jax.experimental.pallas import tpu_sc as plsc

N = 2048
E = 65536
D = 256
HID = 256
HEADS = 4
DH = HID // HEADS
FF = 2048
LAYERS = 2

BD = 1024
GRID = N // BD

WROWS = 64
QUART = N // 4
EDGE_CHUNK = 16384
LANES = 16


def _count_body(src_hbm, dst_hbm, zeros_hbm, out_hbm,
                svm0, dvm0, svm1, dvm1, cbuf, sem0, sem1):
    wid = lax.axis_index("s") * 2 + lax.axis_index("c")
    lo = wid * WROWS
    bufs = [(svm0, dvm0, sem0), (svm1, dvm1, sem1)]
    nch = E // EDGE_CHUNK

    def issue(c):
        sv, dv, sem = bufs[c % 2]
        h1 = pltpu.async_copy(src_hbm.at[pl.ds(c * EDGE_CHUNK, EDGE_CHUNK)],
                              sv, sem)
        h2 = pltpu.async_copy(dst_hbm.at[pl.ds(c * EDGE_CHUNK, EDGE_CHUNK)],
                              dv, sem)
        return h1, h2

    pending = issue(0)
    pltpu.sync_copy(zeros_hbm, cbuf)
    for c in range(nch):
        nxt = issue(c + 1) if c + 1 < nch else None
        for h in pending:
            h.wait()
        sv, dv, _ = bufs[c % 2]

        UNROLL = 8

        def _chunk_body(j, carry):
            base = j * (LANES * UNROLL)
            ds = [dv[pl.ds(base + u * LANES, LANES)] for u in range(UNROLL)]
            ss = [sv[pl.ds(base + u * LANES, LANES)] for u in range(UNROLL)]
            rels = [d - lo for d in ds]
            masks = [(r >= 0) & (r < WROWS) for r in rels]
            cols = [s & (QUART - 1) for s in ss]
            vals = [jnp.full((LANES,), 1, jnp.int32)
                    << ((s >> 9) << 3) for s in ss]
            for u in range(UNROLL):
                plsc.addupdate_scatter(cbuf, [rels[u], cols[u]], vals[u],
                                       mask=masks[u])
            return carry

        lax.fori_loop(0, EDGE_CHUNK // (LANES * UNROLL), _chunk_body, 0)
        pending = nxt

    pltpu.sync_copy(cbuf, out_hbm.at[pl.ds(lo, WROWS)])


@functools.cache
def _count_call():
    return functools.partial(
        pl.kernel,
        out_type=jax.ShapeDtypeStruct((N, QUART), jnp.int32),
        mesh=plsc.VectorSubcoreMesh(core_axis_name="c", subcore_axis_name="s"),
        compiler_params=pltpu.CompilerParams(needs_layout_passes=False),
        scratch_types=[
            pltpu.VMEM((EDGE_CHUNK,), jnp.int32),
            pltpu.VMEM((EDGE_CHUNK,), jnp.int32),
            pltpu.VMEM((EDGE_CHUNK,), jnp.int32),
            pltpu.VMEM((EDGE_CHUNK,), jnp.int32),
            pltpu.VMEM((WROWS, QUART), jnp.int32),
            pltpu.SemaphoreType.DMA,
            pltpu.SemaphoreType.DMA,
        ],
    )(_count_body)


def _proj_core(x, w_ref, asrc_ref, adst_ref, h_ref, esT_ref, ed_ref, es_ref):
    hb = jnp.dot(x, w_ref[...], preferred_element_type=jnp.float32)
    h_ref[...] = hb
    ed_ref[...] = jnp.dot(hb, adst_ref[...], preferred_element_type=jnp.float32)
    es_ref[...] = jnp.dot(hb, asrc_ref[...], preferred_element_type=jnp.float32)
    esT_ref[...] = lax.dot_general(
        asrc_ref[...], hb, (((0,), (1,)), ((), ())),
        preferred_element_type=jnp.float32)


def _proj0_body(x_ref, w_ref, asrc_ref, adst_ref, t_ref, ft_ref, role_ref,
                ca_ref, ef_ref, as3_ref, h_ref, esT_ref, ed_ref, es_ref,
                enc_ref):
    _proj_core(x_ref[...], w_ref, asrc_ref, adst_ref,
               h_ref, esT_ref, ed_ref, es_ref)
    t = t_ref[...]
    ft = ft_ref[0, 0]
    ki = lax.broadcasted_iota(jnp.int32, (BD, HID), 1)
    kf = ki.astype(jnp.float32)
    keven = (ki % 2) == 0
    denom_inv = jnp.exp(kf * (-2.0 * math.log(10000.0) / HID))
    ang = t * denom_inv
    pos = jnp.where(keven, jnp.sin(ang), jnp.cos(ang))
    rc = jnp.clip(role_ref[...], 0, 2)
    renc = jnp.where(rc == 0, ca_ref[...],
                     jnp.where(rc == 1, ef_ref[...], as3_ref[...]))
    enc_ref[...] = (jnp.maximum(t, 0.0) + jnp.maximum(t - ft, 0.0)
                    + pos + renc)


_proj0_call = pl.pallas_call(
    _proj0_body,
    grid=(GRID,),
    in_specs=[
        pl.BlockSpec((BD, HID), lambda i: (i, 0)),
        pl.BlockSpec((HID, HEADS * HID), lambda i: (0, 0)),
        pl.BlockSpec((HEADS * HID, 128), lambda i: (0, 0)),
        pl.BlockSpec((HEADS * HID, 128), lambda i: (0, 0)),
        pl.BlockSpec((BD, 1), lambda i: (i, 0)),
        pl.BlockSpec((1, 1), lambda i: (0, 0)),
        pl.BlockSpec((BD, 1), lambda i: (i, 0)),
        pl.BlockSpec((1, HID), lambda i: (0, 0)),
        pl.BlockSpec((1, HID), lambda i: (0, 0)),
        pl.BlockSpec((1, HID), lambda i: (0, 0)),
    ],
    out_specs=[
        pl.BlockSpec((BD, HEADS * HID), lambda i: (i, 0)),
        pl.BlockSpec((128, BD), lambda i: (0, i)),
        pl.BlockSpec((BD, 128), lambda i: (i, 0)),
        pl.BlockSpec((BD, 128), lambda i: (i, 0)),
        pl.BlockSpec((BD, HID), lambda i: (i, 0)),
    ],
    out_shape=[
        jax.ShapeDtypeStruct((N, HEADS * HID), jnp.float32),
        jax.ShapeDtypeStruct((128, N), jnp.float32),
        jax.ShapeDtypeStruct((N, 128), jnp.float32),
        jax.ShapeDtypeStruct((N, 128), jnp.float32),
        jax.ShapeDtypeStruct((N, HID), jnp.float32),
    ],
)


def _gat_body(ed_ref, es_ref, esT_ref, c_ref, h_ref, b_ref, enc_ref,
              wqkv_ref, bqkv_ref, x1_ref, qkv_ref):
    i = pl.program_id(0)
    packed = c_ref[...]
    cnt = jnp.concatenate(
        [packed & 0xFF, (packed >> 8) & 0xFF, (packed >> 16) & 0xFF,
         lax.shift_right_logical(packed, 24)], axis=1).astype(jnp.float32)
    acc = jnp.zeros((BD, HID), jnp.float32)
    for hd in range(HEADS):
        ecol = ed_ref[:, hd:hd + 1]
        erow = esT_ref[hd:hd + 1, :]
        ea, ea2 = jnp.exp(ecol), jnp.exp(0.2 * ecol)
        eb, eb2 = jnp.exp(erow), jnp.exp(0.2 * erow)
        x_full = jnp.maximum(ea * eb, ea2 * eb2)
        p = cnt * x_full
        rs = jnp.sum(p, axis=1, keepdims=True)
        numer = jnp.dot(p, h_ref[:, hd * HID:(hd + 1) * HID],
                        preferred_element_type=jnp.float32)
        vd = ecol + es_ref[:, hd:hd + 1]
        w = jnp.exp(jnp.maximum(vd, 0.2 * vd))
        hrows = h_ref[pl.ds(i * BD, BD), pl.ds(hd * HID, HID)]
        acc = acc + (numer + w * hrows) / (rs + w)
    x1 = acc * (1.0 / HEADS) + b_ref[...] + enc_ref[...]
    x1_ref[...] = x1
    qkv_ref[...] = (jnp.dot(x1, wqkv_ref[...],
                            preferred_element_type=jnp.float32)
                    + bqkv_ref[...])


_gat_call = pl.pallas_call(
    _gat_body,
    grid=(GRID,),
    in_specs=[
        pl.BlockSpec((BD, 128), lambda i: (i, 0)),
        pl.BlockSpec((BD, 128), lambda i: (i, 0)),
        pl.BlockSpec((128, N), lambda i: (0, 0)),
        pl.BlockSpec((BD, QUART), lambda i: (i, 0)),
        pl.BlockSpec((N, HEADS * HID), lambda i: (0, 0)),
        pl.BlockSpec((1, HID), lambda i: (0, 0)),
        pl.BlockSpec((BD, HID), lambda i: (i, 0)),
        pl.BlockSpec((HID, 3 * HID), lambda i: (0, 0)),
        pl.BlockSpec((1, 3 * HID), lambda i: (0, 0)),
    ],
    out_specs=[
        pl.BlockSpec((BD, HID), lambda i: (i, 0)),
        pl.BlockSpec((BD, 3 * HID), lambda i: (i, 0)),
    ],
    out_shape=[
        jax.ShapeDtypeStruct((N, HID), jnp.float32),
        jax.ShapeDtypeStruct((N, 3 * HID), jnp.float32),
    ],
)


def _layer_norm(y, g, b):
    mu = jnp.mean(y, axis=1, keepdims=True)
    var = jnp.mean((y - mu) ** 2, axis=1, keepdims=True)
    return (y - mu) * lax.rsqrt(var + 1e-5) * g + b


def _make_mha_ff_body(with_proj):
    def body(kv_ref, x_ref, wo_ref, bo_ref, g1_ref, c1_ref,
             w1_ref, b1_ref, w2_ref, b2_ref, g2_ref, c2_ref, *rest):
        i = pl.program_id(0)
        outs = []
        for hd in range(HEADS):
            q = kv_ref[pl.ds(i * BD, BD), pl.ds(hd * DH, DH)]
            k = kv_ref[:, HID + hd * DH:HID + (hd + 1) * DH]
            v = kv_ref[:, 2 * HID + hd * DH:2 * HID + (hd + 1) * DH]
            s = lax.dot_general(q, k, (((1,), (1,)), ((), ())),
                                preferred_element_type=jnp.float32)
            m = jnp.max(s, axis=1, keepdims=True)
            p = jnp.exp(s - m)
            o = jnp.dot(p, v, preferred_element_type=jnp.float32)
            outs.append(o / jnp.sum(p, axis=1, keepdims=True))
        o = jnp.concatenate(outs, axis=1)
        a = (jnp.dot(o, wo_ref[...], preferred_element_type=jnp.float32)
             + bo_ref[...])
        y = _layer_norm(x_ref[...] + a, g1_ref[...], c1_ref[...])
        t = jnp.maximum(
            jnp.dot(y, w1_ref[...], preferred_element_type=jnp.float32)
            + b1_ref[...], 0.0)
        f = (jnp.dot(t, w2_ref[...], preferred_element_type=jnp.float32)
             + b2_ref[...])
        z = _layer_norm(y + f, g2_ref[...], c2_ref[...])
        if with_proj:
            (wg_ref, asrc_ref, adst_ref, x2_ref,
             h_ref, esT_ref, ed_ref, es_ref) = rest
            x2_ref[...] = z
            _proj_core(z, wg_ref, asrc_ref, adst_ref,
                       h_ref, esT_ref, ed_ref, es_ref)
        else:
            (x2_ref,) = rest
            x2_ref[...] = z
    return body


_COMMON_IN = [
    pl.BlockSpec((N, 3 * HID), lambda i: (0, 0)),
    pl.BlockSpec((BD, HID), lambda i: (i, 0)),
    pl.BlockSpec((HID, HID), lambda i: (0, 0)),
    pl.BlockSpec((1, HID), lambda i: (0, 0)),
    pl.BlockSpec((1, HID), lambda i: (0, 0)),
    pl.BlockSpec((1, HID), lambda i: (0, 0)),
    pl.BlockSpec((HID, FF), lambda i: (0, 0)),
    pl.BlockSpec((1, FF), lambda i: (0, 0)),
    pl.BlockSpec((FF, HID), lambda i: (0, 0)),
    pl.BlockSpec((1, HID), lambda i: (0, 0)),
    pl.BlockSpec((1, HID), lambda i: (0, 0)),
    pl.BlockSpec((1, HID), lambda i: (0, 0)),
]

_mha_ff_proj_call = pl.pallas_call(
    _make_mha_ff_body(True),
    grid=(GRID,),
    in_specs=_COMMON_IN + [
        pl.BlockSpec((HID, HEADS * HID), lambda i: (0, 0)),
        pl.BlockSpec((HEADS * HID, 128), lambda i: (0, 0)),
        pl.BlockSpec((HEADS * HID, 128), lambda i: (0, 0)),
    ],
    out_specs=[
        pl.BlockSpec((BD, HID), lambda i: (i, 0)),
        pl.BlockSpec((BD, HEADS * HID), lambda i: (i, 0)),
        pl.BlockSpec((128, BD), lambda i: (0, i)),
        pl.BlockSpec((BD, 128), lambda i: (i, 0)),
        pl.BlockSpec((BD, 128), lambda i: (i, 0)),
    ],
    out_shape=[
        jax.ShapeDtypeStruct((N, HID), jnp.float32),
        jax.ShapeDtypeStruct((N, HEADS * HID), jnp.float32),
        jax.ShapeDtypeStruct((128, N), jnp.float32),
        jax.ShapeDtypeStruct((N, 128), jnp.float32),
        jax.ShapeDtypeStruct((N, 128), jnp.float32),
    ],
)

_mha_ff_call = pl.pallas_call(
    _make_mha_ff_body(False),
    grid=(GRID,),
    in_specs=_COMMON_IN,
    out_specs=[pl.BlockSpec((BD, HID), lambda i: (i, 0))],
    out_shape=[jax.ShapeDtypeStruct((N, HID), jnp.float32)],
)


def _blockdiag(a):
    eye = jnp.eye(HEADS, 128, dtype=a.dtype)
    return (a[:, :, None] * eye[:, None, :]).reshape(HEADS * HID, 128)


def kernel(x, edge_index, timestamps, first_timestamp, event_roles, params):
    src = edge_index[0]
    dst = edge_index[1]
    zeros_tile = jnp.zeros((WROWS, QUART), jnp.int32)
    c_mat = _count_call()(src, dst, zeros_tile)

    def wqkv_b(i):
        sc = 1.0 / math.sqrt(DH)
        w = jnp.concatenate([params[f'Wq{i}'] * sc, params[f'Wk{i}'],
                             params[f'Wv{i}']], axis=1)
        b = jnp.concatenate([params[f'bq{i}'] * sc, params[f'bk{i}'],
                             params[f'bv{i}']])[None]
        return w, b

    h, esT, ed, es, enc = _proj0_call(
        x, params['W_gat0'], _blockdiag(params['a_src0']),
        _blockdiag(params['a_dst0']), timestamps[:, None],
        first_timestamp[:, None], event_roles[:, None],
        params['cause'][None], params['effect'][None], params['assoc'][None])

    xcur = None
    for i in range(LAYERS):
        wq, bq = wqkv_b(i)
        x1, qkv = _gat_call(ed, es, esT, c_mat, h, params[f'b_gat{i}'][None],
                            enc, wq, bq)
        common = (qkv, x1, params[f'Wo{i}'], params[f'bo{i}'][None],
                  params[f'ln1_g{i}'][None], params[f'ln1_b{i}'][None],
                  params[f'W1{i}'], params[f'b1{i}'][None], params[f'W2{i}'],
                  params[f'b2{i}'][None], params[f'ln2_g{i}'][None],
                  params[f'ln2_b{i}'][None])
        if i + 1 < LAYERS:
            xcur, h, esT, ed, es = _mha_ff_proj_call(
                *common, params[f'W_gat{i + 1}'],
                _blockdiag(params[f'a_src{i + 1}']),
                _blockdiag(params[f'a_dst{i + 1}']))
        else:
            (xcur,) = _mha_ff_call(*common)
    return xcur


# speedup vs baseline: 1.0114x; 1.0114x over previous
import functools
import math

import jax
import jax.numpy as jnp
from jax import lax
from jax.experimental import pallas as pl
from jax.experimental.pallas import tpu as pltpu
from jax.experimental.pallas import tpu_sc as plsc

N = 2048
E = 65536
D = 256
HID = 256
HEADS = 4
DH = HID // HEADS
FF = 2048
LAYERS = 2

BD = 512
GRID = N // BD

WROWS = 64
QUART = N // 4
EDGE_CHUNK = 16384
LANES = 16


def _count_body(src_hbm, dst_hbm, zeros_hbm, out_hbm,
                svm0, dvm0, svm1, dvm1, cbuf, sem0, sem1):
    wid = lax.axis_index("s") * 2 + lax.axis_index("c")
    lo = wid * WROWS
    bufs = [(svm0, dvm0, sem0), (svm1, dvm1, sem1)]
    nch = E // EDGE_CHUNK

    def issue(c):
        sv, dv, sem = bufs[c % 2]
        h1 = pltpu.async_copy(src_hbm.at[pl.ds(c * EDGE_CHUNK, EDGE_CHUNK)],
                              sv, sem)
        h2 = pltpu.async_copy(dst_hbm.at[pl.ds(c * EDGE_CHUNK, EDGE_CHUNK)],
                              dv, sem)
        return h1, h2

    pending = issue(0)
    pltpu.sync_copy(zeros_hbm, cbuf)
    for c in range(nch):
        nxt = issue(c + 1) if c + 1 < nch else None
        for h in pending:
            h.wait()
        sv, dv, _ = bufs[c % 2]

        UNROLL = 8

        def _chunk_body(j, carry):
            base = j * (LANES * UNROLL)
            ds = [dv[pl.ds(base + u * LANES, LANES)] for u in range(UNROLL)]
            ss = [sv[pl.ds(base + u * LANES, LANES)] for u in range(UNROLL)]
            rels = [d - lo for d in ds]
            masks = [(r >= 0) & (r < WROWS) for r in rels]
            cols = [s & (QUART - 1) for s in ss]
            vals = [jnp.full((LANES,), 1, jnp.int32)
                    << ((s >> 9) << 3) for s in ss]
            for u in range(UNROLL):
                plsc.addupdate_scatter(cbuf, [rels[u], cols[u]], vals[u],
                                       mask=masks[u])
            return carry

        lax.fori_loop(0, EDGE_CHUNK // (LANES * UNROLL), _chunk_body, 0)
        pending = nxt

    pltpu.sync_copy(cbuf, out_hbm.at[pl.ds(lo, WROWS)])


@functools.cache
def _count_call():
    return functools.partial(
        pl.kernel,
        out_type=jax.ShapeDtypeStruct((N, QUART), jnp.int32),
        mesh=plsc.VectorSubcoreMesh(core_axis_name="c", subcore_axis_name="s"),
        compiler_params=pltpu.CompilerParams(needs_layout_passes=False),
        scratch_types=[
            pltpu.VMEM((EDGE_CHUNK,), jnp.int32),
            pltpu.VMEM((EDGE_CHUNK,), jnp.int32),
            pltpu.VMEM((EDGE_CHUNK,), jnp.int32),
            pltpu.VMEM((EDGE_CHUNK,), jnp.int32),
            pltpu.VMEM((WROWS, QUART), jnp.int32),
            pltpu.SemaphoreType.DMA,
            pltpu.SemaphoreType.DMA,
        ],
    )(_count_body)


def _proj_core(x, w_ref, asrc_ref, adst_ref, h_ref, esT_ref, ed_ref, es_ref):
    hb = jnp.dot(x, w_ref[...], preferred_element_type=jnp.float32)
    h_ref[...] = hb
    ed_ref[...] = jnp.dot(hb, adst_ref[...], preferred_element_type=jnp.float32)
    es_ref[...] = jnp.dot(hb, asrc_ref[...], preferred_element_type=jnp.float32)
    esT_ref[...] = lax.dot_general(
        asrc_ref[...], hb, (((0,), (1,)), ((), ())),
        preferred_element_type=jnp.float32)


def _proj0_body(x_ref, w_ref, asrc_ref, adst_ref, t_ref, ft_ref, role_ref,
                ca_ref, ef_ref, as3_ref, h_ref, esT_ref, ed_ref, es_ref,
                enc_ref):
    _proj_core(x_ref[...], w_ref, asrc_ref, adst_ref,
               h_ref, esT_ref, ed_ref, es_ref)
    t = t_ref[...]
    ft = ft_ref[0, 0]
    ki = lax.broadcasted_iota(jnp.int32, (BD, HID), 1)
    kf = ki.astype(jnp.float32)
    keven = (ki % 2) == 0
    denom_inv = jnp.exp(kf * (-2.0 * math.log(10000.0) / HID))
    ang = t * denom_inv
    pos = jnp.where(keven, jnp.sin(ang), jnp.cos(ang))
    rc = jnp.clip(role_ref[...], 0, 2)
    renc = jnp.where(rc == 0, ca_ref[...],
                     jnp.where(rc == 1, ef_ref[...], as3_ref[...]))
    enc_ref[...] = (jnp.maximum(t, 0.0) + jnp.maximum(t - ft, 0.0)
                    + pos + renc)


_proj0_call = pl.pallas_call(
    _proj0_body,
    grid=(GRID,),
    in_specs=[
        pl.BlockSpec((BD, HID), lambda i: (i, 0)),
        pl.BlockSpec((HID, HEADS * HID), lambda i: (0, 0)),
        pl.BlockSpec((HEADS * HID, 128), lambda i: (0, 0)),
        pl.BlockSpec((HEADS * HID, 128), lambda i: (0, 0)),
        pl.BlockSpec((BD, 1), lambda i: (i, 0)),
        pl.BlockSpec((1, 1), lambda i: (0, 0)),
        pl.BlockSpec((BD, 1), lambda i: (i, 0)),
        pl.BlockSpec((1, HID), lambda i: (0, 0)),
        pl.BlockSpec((1, HID), lambda i: (0, 0)),
        pl.BlockSpec((1, HID), lambda i: (0, 0)),
    ],
    out_specs=[
        pl.BlockSpec((BD, HEADS * HID), lambda i: (i, 0)),
        pl.BlockSpec((128, BD), lambda i: (0, i)),
        pl.BlockSpec((BD, 128), lambda i: (i, 0)),
        pl.BlockSpec((BD, 128), lambda i: (i, 0)),
        pl.BlockSpec((BD, HID), lambda i: (i, 0)),
    ],
    out_shape=[
        jax.ShapeDtypeStruct((N, HEADS * HID), jnp.float32),
        jax.ShapeDtypeStruct((128, N), jnp.float32),
        jax.ShapeDtypeStruct((N, 128), jnp.float32),
        jax.ShapeDtypeStruct((N, 128), jnp.float32),
        jax.ShapeDtypeStruct((N, HID), jnp.float32),
    ],
)


def _gat_body(ed_ref, es_ref, esT_ref, c_ref, h_ref, b_ref, enc_ref,
              wqkv_ref, bqkv_ref, x1_ref, qkv_ref):
    i = pl.program_id(0)
    packed = c_ref[...]
    cnt = jnp.concatenate(
        [packed & 0xFF, (packed >> 8) & 0xFF, (packed >> 16) & 0xFF,
         lax.shift_right_logical(packed, 24)], axis=1).astype(jnp.float32)
    acc = jnp.zeros((BD, HID), jnp.float32)
    for hd in range(HEADS):
        ecol = ed_ref[:, hd:hd + 1]
        erow = esT_ref[hd:hd + 1, :]
        ea, ea2 = jnp.exp(ecol), jnp.exp(0.2 * ecol)
        eb, eb2 = jnp.exp(erow), jnp.exp(0.2 * erow)
        x_full = jnp.maximum(ea * eb, ea2 * eb2)
        p = cnt * x_full
        rs = jnp.sum(p, axis=1, keepdims=True)
        numer = jnp.dot(p, h_ref[:, hd * HID:(hd + 1) * HID],
                        preferred_element_type=jnp.float32)
        vd = ecol + es_ref[:, hd:hd + 1]
        w = jnp.exp(jnp.maximum(vd, 0.2 * vd))
        hrows = h_ref[pl.ds(i * BD, BD), pl.ds(hd * HID, HID)]
        acc = acc + (numer + w * hrows) / (rs + w)
    x1 = acc * (1.0 / HEADS) + b_ref[...] + enc_ref[...]
    x1_ref[...] = x1
    qkv_ref[...] = (jnp.dot(x1, wqkv_ref[...],
                            preferred_element_type=jnp.float32)
                    + bqkv_ref[...])


_gat_call = pl.pallas_call(
    _gat_body,
    grid=(GRID,),
    in_specs=[
        pl.BlockSpec((BD, 128), lambda i: (i, 0)),
        pl.BlockSpec((BD, 128), lambda i: (i, 0)),
        pl.BlockSpec((128, N), lambda i: (0, 0)),
        pl.BlockSpec((BD, QUART), lambda i: (i, 0)),
        pl.BlockSpec((N, HEADS * HID), lambda i: (0, 0)),
        pl.BlockSpec((1, HID), lambda i: (0, 0)),
        pl.BlockSpec((BD, HID), lambda i: (i, 0)),
        pl.BlockSpec((HID, 3 * HID), lambda i: (0, 0)),
        pl.BlockSpec((1, 3 * HID), lambda i: (0, 0)),
    ],
    out_specs=[
        pl.BlockSpec((BD, HID), lambda i: (i, 0)),
        pl.BlockSpec((BD, 3 * HID), lambda i: (i, 0)),
    ],
    out_shape=[
        jax.ShapeDtypeStruct((N, HID), jnp.float32),
        jax.ShapeDtypeStruct((N, 3 * HID), jnp.float32),
    ],
)


def _layer_norm(y, g, b):
    mu = jnp.mean(y, axis=1, keepdims=True)
    var = jnp.mean((y - mu) ** 2, axis=1, keepdims=True)
    return (y - mu) * lax.rsqrt(var + 1e-5) * g + b


def _make_mha_ff_body(with_proj):
    def body(kv_ref, x_ref, wo_ref, bo_ref, g1_ref, c1_ref,
             w1_ref, b1_ref, w2_ref, b2_ref, g2_ref, c2_ref, *rest):
        i = pl.program_id(0)
        outs = []
        for hd in range(HEADS):
            q = kv_ref[pl.ds(i * BD, BD), pl.ds(hd * DH, DH)]
            k = kv_ref[:, HID + hd * DH:HID + (hd + 1) * DH]
            v = kv_ref[:, 2 * HID + hd * DH:2 * HID + (hd + 1) * DH]
            s = lax.dot_general(q, k, (((1,), (1,)), ((), ())),
                                preferred_element_type=jnp.float32)
            m = jnp.max(s, axis=1, keepdims=True)
            p = jnp.exp(s - m)
            o = jnp.dot(p, v, preferred_element_type=jnp.float32)
            outs.append(o / jnp.sum(p, axis=1, keepdims=True))
        o = jnp.concatenate(outs, axis=1)
        a = (jnp.dot(o, wo_ref[...], preferred_element_type=jnp.float32)
             + bo_ref[...])
        y = _layer_norm(x_ref[...] + a, g1_ref[...], c1_ref[...])
        t = jnp.maximum(
            jnp.dot(y, w1_ref[...], preferred_element_type=jnp.float32)
            + b1_ref[...], 0.0)
        f = (jnp.dot(t, w2_ref[...], preferred_element_type=jnp.float32)
             + b2_ref[...])
        z = _layer_norm(y + f, g2_ref[...], c2_ref[...])
        if with_proj:
            (wg_ref, asrc_ref, adst_ref, x2_ref,
             h_ref, esT_ref, ed_ref, es_ref) = rest
            x2_ref[...] = z
            _proj_core(z, wg_ref, asrc_ref, adst_ref,
                       h_ref, esT_ref, ed_ref, es_ref)
        else:
            (x2_ref,) = rest
            x2_ref[...] = z
    return body


_COMMON_IN = [
    pl.BlockSpec((N, 3 * HID), lambda i: (0, 0)),
    pl.BlockSpec((BD, HID), lambda i: (i, 0)),
    pl.BlockSpec((HID, HID), lambda i: (0, 0)),
    pl.BlockSpec((1, HID), lambda i: (0, 0)),
    pl.BlockSpec((1, HID), lambda i: (0, 0)),
    pl.BlockSpec((1, HID), lambda i: (0, 0)),
    pl.BlockSpec((HID, FF), lambda i: (0, 0)),
    pl.BlockSpec((1, FF), lambda i: (0, 0)),
    pl.BlockSpec((FF, HID), lambda i: (0, 0)),
    pl.BlockSpec((1, HID), lambda i: (0, 0)),
    pl.BlockSpec((1, HID), lambda i: (0, 0)),
    pl.BlockSpec((1, HID), lambda i: (0, 0)),
]

_mha_ff_proj_call = pl.pallas_call(
    _make_mha_ff_body(True),
    grid=(GRID,),
    in_specs=_COMMON_IN + [
        pl.BlockSpec((HID, HEADS * HID), lambda i: (0, 0)),
        pl.BlockSpec((HEADS * HID, 128), lambda i: (0, 0)),
        pl.BlockSpec((HEADS * HID, 128), lambda i: (0, 0)),
    ],
    out_specs=[
        pl.BlockSpec((BD, HID), lambda i: (i, 0)),
        pl.BlockSpec((BD, HEADS * HID), lambda i: (i, 0)),
        pl.BlockSpec((128, BD), lambda i: (0, i)),
        pl.BlockSpec((BD, 128), lambda i: (i, 0)),
        pl.BlockSpec((BD, 128), lambda i: (i, 0)),
    ],
    out_shape=[
        jax.ShapeDtypeStruct((N, HID), jnp.float32),
        jax.ShapeDtypeStruct((N, HEADS * HID), jnp.float32),
        jax.ShapeDtypeStruct((128, N), jnp.float32),
        jax.ShapeDtypeStruct((N, 128), jnp.float32),
        jax.ShapeDtypeStruct((N, 128), jnp.float32),
    ],
)

_mha_ff_call = pl.pallas_call(
    _make_mha_ff_body(False),
    grid=(GRID,),
    in_specs=_COMMON_IN,
    out_specs=[pl.BlockSpec((BD, HID), lambda i: (i, 0))],
    out_shape=[jax.ShapeDtypeStruct((N, HID), jnp.float32)],
)


def _blockdiag(a):
    eye = jnp.eye(HEADS, 128, dtype=a.dtype)
    return (a[:, :, None] * eye[:, None, :]).reshape(HEADS * HID, 128)


def kernel(x, edge_index, timestamps, first_timestamp, event_roles, params):
    src = edge_index[0]
    dst = edge_index[1]
    zeros_tile = jnp.zeros((WROWS, QUART), jnp.int32)
    c_mat = _count_call()(src, dst, zeros_tile)

    def wqkv_b(i):
        sc = 1.0 / math.sqrt(DH)
        w = jnp.concatenate([params[f'Wq{i}'] * sc, params[f'Wk{i}'],
                             params[f'Wv{i}']], axis=1)
        b = jnp.concatenate([params[f'bq{i}'] * sc, params[f'bk{i}'],
                             params[f'bv{i}']])[None]
        return w, b

    h, esT, ed, es, enc = _proj0_call(
        x, params['W_gat0'], _blockdiag(params['a_src0']),
        _blockdiag(params['a_dst0']), timestamps[:, None],
        first_timestamp[:, None], event_roles[:, None],
        params['cause'][None], params['effect'][None], params['assoc'][None])

    xcur = None
    for i in range(LAYERS):
        wq, bq = wqkv_b(i)
        x1, qkv = _gat_call(ed, es, esT, c_mat, h, params[f'b_gat{i}'][None],
                            enc, wq, bq)
        common = (qkv, x1, params[f'Wo{i}'], params[f'bo{i}'][None],
                  params[f'ln1_g{i}'][None], params[f'ln1_b{i}'][None],
                  params[f'W1{i}'], params[f'b1{i}'][None], params[f'W2{i}'],
                  params[f'b2{i}'][None], params[f'ln2_g{i}'][None],
                  params[f'ln2_b{i}'][None])
        if i + 1 < LAYERS:
            xcur, h, esT, ed, es = _mha_ff_proj_call(
                *common, params[f'W_gat{i + 1}'],
                _blockdiag(params[f'a_src{i + 1}']),
                _blockdiag(params[f'a_dst{i + 1}']))
        else:
            (xcur,) = _mha_ff_call(*common)
    return xcur
